# baseline (speedup 1.0000x reference)
import jax
import jax.numpy as jnp
from jax import lax
from jax.experimental import pallas as pl
from jax.experimental.pallas import tpu as pltpu
from jax.experimental.pallas import tpu_sc as plsc

_CYCLE = 168
_LEN = 336
_CH = 256
_B = 1024
_NC = 2
_NS = 16
_NW = _NC * _NS
_BPW = _B // _NW


def _sc_body(idx_hbm, data_hbm, out_hbm, idx_v, ddd_sh, sem):
    cid = lax.axis_index("c")
    sid = lax.axis_index("s")
    wid = sid * _NC + cid
    base = wid * _BPW
    pltpu.sync_copy(idx_hbm.at[pl.ds(base, _BPW)], idx_v)
    @pl.when(sid == 0)
    def _stage():
        pltpu.sync_copy(data_hbm, ddd_sh.at[pl.ds(0, _CYCLE)])
        pltpu.sync_copy(data_hbm, ddd_sh.at[pl.ds(_CYCLE, _CYCLE)])
        pltpu.sync_copy(data_hbm, ddd_sh.at[pl.ds(2 * _CYCLE, _CYCLE)])
    plsc.subcore_barrier()
    copies = []
    for g in range(_BPW // 16):
        vec = idx_v[pl.ds(g * 16, 16)]
        for j in range(16):
            b = g * 16 + j
            i = vec[j]
            copies.append(
                pltpu.async_copy(ddd_sh.at[pl.ds(i, _LEN)], out_hbm.at[base + b], sem)
            )
    for c in copies:
        c.wait()


def kernel(index, length, data):
    start = jnp.mod(index.astype(jnp.int32) + (length - _LEN), _CYCLE)
    start = start.astype(jnp.int32)
    mesh = plsc.VectorSubcoreMesh(core_axis_name="c", subcore_axis_name="s")
    k = pl.kernel(
        _sc_body,
        out_type=jax.ShapeDtypeStruct((_B, _LEN, _CH), jnp.float32),
        mesh=mesh,
        scratch_types=[
            pltpu.VMEM((_BPW,), jnp.int32),
            pltpu.VMEM_SHARED((3 * _CYCLE, _CH), jnp.float32),
            pltpu.SemaphoreType.DMA,
        ],
        compiler_params=pltpu.CompilerParams(use_tc_tiling_on_sc=False),
    )
    return k(start, data)


# speedup vs baseline: 1.1094x; 1.1094x over previous
import jax
import jax.numpy as jnp
from jax import lax
from jax.experimental import pallas as pl
from jax.experimental.pallas import tpu as pltpu
from jax.experimental.pallas import tpu_sc as plsc

_CYCLE = 168
_LEN = 336
_CH = 256
_B = 1024
_NC = 2
_NS = 16
_NW = _NC * _NS
_BPW = _B // _NW


def _sc_body(idx_hbm, data_hbm, out_hbm, idx_v, dd_v, ddd_sh, sem, stage_sem):
    cid = lax.axis_index("c")
    sid = lax.axis_index("s")
    wid = sid * _NC + cid
    base = wid * _BPW
    stage = [
        pltpu.async_copy(idx_hbm.at[pl.ds(base, _BPW)], idx_v, stage_sem),
        pltpu.async_copy(data_hbm, dd_v.at[pl.ds(0, _CYCLE)], stage_sem),
        pltpu.async_copy(data_hbm, dd_v.at[pl.ds(_CYCLE, _CYCLE)], stage_sem),
    ]
    for c in stage:
        c.wait()

    @pl.when(sid == 0)
    def _stage_shared():
        shared = [
            pltpu.async_copy(data_hbm, ddd_sh.at[pl.ds(0, _CYCLE)], stage_sem),
            pltpu.async_copy(data_hbm, ddd_sh.at[pl.ds(_CYCLE, _CYCLE)], stage_sem),
            pltpu.async_copy(data_hbm, ddd_sh.at[pl.ds(2 * _CYCLE, _CYCLE)], stage_sem),
        ]
        for c in shared:
            c.wait()

    plsc.subcore_barrier()
    copies = []
    for g in range(_BPW // 16):
        vec = idx_v[pl.ds(g * 16, 16)]
        for j in range(16):
            b = g * 16 + j
            i = vec[j]
            if b % 2 == 0:
                copies.append(pltpu.async_copy(
                    dd_v.at[pl.ds(i, _CYCLE)],
                    out_hbm.at[base + b, pl.ds(0, _CYCLE)], sem))
                copies.append(pltpu.async_copy(
                    dd_v.at[pl.ds(i, _CYCLE)],
                    out_hbm.at[base + b, pl.ds(_CYCLE, _CYCLE)], sem))
            else:
                copies.append(pltpu.async_copy(
                    ddd_sh.at[pl.ds(i, _LEN)], out_hbm.at[base + b], sem))
    for c in copies:
        c.wait()


def kernel(index, length, data):
    start = jnp.mod(index.astype(jnp.int32) + (length - _LEN), _CYCLE)
    start = start.astype(jnp.int32)
    mesh = plsc.VectorSubcoreMesh(core_axis_name="c", subcore_axis_name="s")
    k = pl.kernel(
        _sc_body,
        out_type=jax.ShapeDtypeStruct((_B, _LEN, _CH), jnp.float32),
        mesh=mesh,
        scratch_types=[
            pltpu.VMEM((_BPW,), jnp.int32),
            pltpu.VMEM((2 * _CYCLE, _CH), jnp.float32),
            pltpu.VMEM_SHARED((3 * _CYCLE, _CH), jnp.float32),
            pltpu.SemaphoreType.DMA,
            pltpu.SemaphoreType.DMA,
        ],
        compiler_params=pltpu.CompilerParams(use_tc_tiling_on_sc=False),
    )
    return k(start, data)


# speedup vs baseline: 1.4427x; 1.3005x over previous
import jax
import jax.numpy as jnp
from jax import lax
from jax.experimental import pallas as pl
from jax.experimental.pallas import tpu as pltpu

_CYCLE = 168
_LEN = 336
_CH = 256
_B = 1024


def _tc_body(s_ref, data_ref, out_ref, quad, rots):
    pid = pl.program_id(0)

    @pl.when(pid == 0)
    def _build():
        for k in range(4):
            quad[pl.ds(k * _CYCLE, _CYCLE), :] = data_ref[...]
        for r in range(8):
            rots[r] = quad[pl.ds(r, 3 * _CYCLE), :]

    i = s_ref[pid]
    r = lax.rem(i, 8)
    off = pl.multiple_of(i - r, 8)
    out_ref[0] = rots[r, pl.ds(off, _LEN), :]


def tc_kernel(start, data):
    grid_spec = pltpu.PrefetchScalarGridSpec(
        num_scalar_prefetch=1,
        grid=(_B,),
        in_specs=[pl.BlockSpec((_CYCLE, _CH), lambda b, s: (0, 0))],
        out_specs=pl.BlockSpec((1, _LEN, _CH), lambda b, s: (b, 0, 0)),
        scratch_shapes=[
            pltpu.VMEM((4 * _CYCLE, _CH), jnp.float32),
            pltpu.VMEM((8, 3 * _CYCLE, _CH), jnp.float32),
        ],
    )
    return pl.pallas_call(
        _tc_body,
        grid_spec=grid_spec,
        out_shape=jax.ShapeDtypeStruct((_B, _LEN, _CH), jnp.float32),
    )(start, data)


def kernel(index, length, data):
    start = jnp.mod(index.astype(jnp.int32) + (length - _LEN), _CYCLE)
    start = start.astype(jnp.int32)
    return tc_kernel(start, data)


# speedup vs baseline: 3.6707x; 2.5443x over previous
import jax
import jax.numpy as jnp
from jax import lax
from jax.experimental import pallas as pl
from jax.experimental.pallas import tpu as pltpu

_CYCLE = 168
_LEN = 336
_CH = 256
_B = 1024


def _tc_body(s_ref, data_ref, out_ref, quad, rots):
    pid = pl.program_id(0)

    @pl.when(pid == 0)
    def _build():
        for k in range(4):
            quad[pl.ds(k * _CYCLE, _CYCLE), :] = data_ref[...]
        for r in range(8):
            rots[r] = quad[pl.ds(r, 3 * _CYCLE), :]

    for k in range(4):
        i = s_ref[pid * 4 + k]
        r = lax.rem(i, 8)
        off = pl.multiple_of(i - r, 8)
        out_ref[k] = rots[r, pl.ds(off, _LEN), :]


def tc_kernel(start, data):
    grid_spec = pltpu.PrefetchScalarGridSpec(
        num_scalar_prefetch=1,
        grid=(_B // 4,),
        in_specs=[pl.BlockSpec((_CYCLE, _CH), lambda b, s: (0, 0))],
        out_specs=pl.BlockSpec((4, _LEN, _CH), lambda b, s: (b, 0, 0)),
        scratch_shapes=[
            pltpu.VMEM((4 * _CYCLE, _CH), jnp.float32),
            pltpu.VMEM((8, 3 * _CYCLE, _CH), jnp.float32),
        ],
    )
    return pl.pallas_call(
        _tc_body,
        grid_spec=grid_spec,
        out_shape=jax.ShapeDtypeStruct((_B, _LEN, _CH), jnp.float32),
    )(start, data)


def kernel(index, length, data):
    start = jnp.mod(index.astype(jnp.int32) + (length - _LEN), _CYCLE)
    start = start.astype(jnp.int32)
    return tc_kernel(start, data)
